# Initial kernel scaffold; baseline (speedup 1.0000x reference)
import jax
import jax.numpy as jnp
from jax.experimental import pallas as pl


def kernel(indices, vals):
    raise NotImplementedError("write your pallas kernel here")


# speedup vs baseline: 3.1252x; 3.1252x over previous
import functools

import jax
import jax.numpy as jnp
from jax import lax
from jax.experimental import pallas as pl
from jax.experimental.pallas import tpu as pltpu
from jax.experimental.pallas import tpu_sc as plsc

_LANES = 16
_NUM_CORES = 2
_NUM_SUBCORES = 16
_NUM_WORKERS = _NUM_CORES * _NUM_SUBCORES


def _chunk_offsets(n):
    offs = list(range(0, n - _LANES + 1, _LANES))
    if offs[-1] + _LANES < n:
        offs.append(n - _LANES)
    return offs


def _sc_multihot(indices, vals):
    B, H = indices.shape
    V = 100000
    rows_per_w = B // _NUM_WORKERS
    offs = _chunk_offsets(H)
    mesh = plsc.VectorSubcoreMesh(core_axis_name="c", subcore_axis_name="s")

    @functools.partial(
        pl.kernel,
        out_type=jax.ShapeDtypeStruct((B, V), jnp.float32),
        mesh=mesh,
        scratch_types=[
            pltpu.VMEM((V,), jnp.float32),
            pltpu.VMEM((H,), jnp.int32),
            pltpu.VMEM((H,), jnp.float32),
        ],
        compiler_params=pltpu.CompilerParams(needs_layout_passes=False),
    )
    def body(idx_hbm, vals_hbm, out_hbm, rowbuf, idx_v, val_v):
        wid = lax.axis_index("s") * _NUM_CORES + lax.axis_index("c")
        zero16 = jnp.zeros((_LANES,), jnp.float32)

        unroll = 10
        span = _LANES * unroll

        def zero_body(i, carry):
            base = i * span
            for u in range(unroll):
                rowbuf[pl.ds(base + u * _LANES, _LANES)] = zero16
            return carry

        lax.fori_loop(0, V // span, zero_body, 0)

        def row_body(r, carry):
            row = wid * rows_per_w + r
            pltpu.sync_copy(idx_hbm.at[row], idx_v)
            pltpu.sync_copy(vals_hbm.at[row], val_v)
            for o in offs:
                iv = idx_v[pl.ds(o, _LANES)]
                vv = val_v[pl.ds(o, _LANES)]
                plsc.store_scatter(rowbuf, [iv], vv)
            pltpu.sync_copy(rowbuf, out_hbm.at[row])
            for o in offs:
                iv = idx_v[pl.ds(o, _LANES)]
                plsc.store_scatter(rowbuf, [iv], zero16)
            return carry

        lax.fori_loop(0, rows_per_w, row_body, 0)

    return body(indices, vals)


def kernel(indices, vals):
    B = indices.shape[0]
    encoded = _sc_multihot(indices, vals)
    mask = jnp.ones((B, 1), dtype=jnp.int32)
    return encoded[:, None, :], mask


# speedup vs baseline: 3.3535x; 1.0731x over previous
import functools

import jax
import jax.numpy as jnp
from jax import lax
from jax.experimental import pallas as pl
from jax.experimental.pallas import tpu as pltpu
from jax.experimental.pallas import tpu_sc as plsc

_LANES = 16
_NUM_CORES = 2
_NUM_SUBCORES = 16
_NUM_WORKERS = _NUM_CORES * _NUM_SUBCORES


def _chunk_offsets(n):
    offs = list(range(0, n - _LANES + 1, _LANES))
    if offs[-1] + _LANES < n:
        offs.append(n - _LANES)
    return offs


def _sc_multihot(indices, vals):
    B, H = indices.shape
    V = 100000
    LO = 49920
    HI = V - LO
    rows_per_w = B // _NUM_WORKERS
    offs = _chunk_offsets(H)
    mesh = plsc.VectorSubcoreMesh(core_axis_name="c", subcore_axis_name="s")

    @functools.partial(
        pl.kernel,
        out_type=jax.ShapeDtypeStruct((B, V), jnp.float32),
        mesh=mesh,
        scratch_types=[
            pltpu.VMEM((LO,), jnp.float32),
            pltpu.VMEM((HI,), jnp.float32),
            pltpu.VMEM((rows_per_w, H), jnp.int32),
            pltpu.VMEM((rows_per_w, H), jnp.float32),
            pltpu.SemaphoreType.DMA,
            pltpu.SemaphoreType.DMA,
        ],
        compiler_params=pltpu.CompilerParams(needs_layout_passes=False),
    )
    def body(idx_hbm, vals_hbm, out_hbm, buf_lo, buf_hi, idx_blk, val_blk,
             sem_lo, sem_hi):
        wid = lax.axis_index("s") * _NUM_CORES + lax.axis_index("c")
        base_row = wid * rows_per_w
        zero16 = jnp.zeros((_LANES,), jnp.float32)
        half = jnp.int32(LO)

        pltpu.sync_copy(idx_hbm.at[pl.ds(base_row, rows_per_w)], idx_blk)
        pltpu.sync_copy(vals_hbm.at[pl.ds(base_row, rows_per_w)], val_blk)

        unroll = 10
        span = _LANES * unroll

        def zero_body(i, carry):
            base = i * span
            for u in range(unroll):
                off = pl.ds(base + u * _LANES, _LANES)
                buf_lo[off] = zero16
                buf_hi[off] = zero16
            return carry

        nz = min(LO, HI)
        lax.fori_loop(0, nz // span, zero_body, 0, unroll=False)
        for buf, n in ((buf_lo, LO), (buf_hi, HI)):
            rem = n - (nz // span) * span
            for u in range(rem // _LANES):
                buf[pl.ds(n - rem + u * _LANES, _LANES)] = zero16

        def scatter_half(r, lo):
            buf = buf_lo if lo else buf_hi
            for o in offs:
                iv = idx_blk[r, pl.ds(o, _LANES)]
                vv = val_blk[r, pl.ds(o, _LANES)]
                m = iv < half if lo else iv >= half
                plsc.store_scatter(buf, [iv if lo else iv - half], vv, mask=m)

        def unscatter_half(r, lo):
            buf = buf_lo if lo else buf_hi
            for o in offs:
                iv = idx_blk[r, pl.ds(o, _LANES)]
                m = iv < half if lo else iv >= half
                plsc.store_scatter(buf, [iv if lo else iv - half], zero16,
                                   mask=m)

        def dma_lo(row):
            return pltpu.make_async_copy(
                buf_lo, out_hbm.at[row].at[pl.ds(0, LO)], sem_lo)

        def dma_hi(row):
            return pltpu.make_async_copy(
                buf_hi, out_hbm.at[row].at[pl.ds(LO, HI)], sem_hi)

        def row_body(r, carry):
            row = base_row + r

            @pl.when(r > 0)
            def _():
                dma_lo(row).wait()
                unscatter_half(r - 1, True)

            scatter_half(r, True)
            dma_lo(row).start()

            @pl.when(r > 0)
            def _():
                dma_hi(row).wait()
                unscatter_half(r - 1, False)

            scatter_half(r, False)
            dma_hi(row).start()
            return carry

        lax.fori_loop(0, rows_per_w, row_body, 0, unroll=False)
        dma_lo(base_row).wait()
        dma_hi(base_row).wait()

    return body(indices, vals)


def kernel(indices, vals):
    B = indices.shape[0]
    encoded = _sc_multihot(indices, vals)
    mask = jnp.ones((B, 1), dtype=jnp.int32)
    return encoded[:, None, :], mask
